# baseline (speedup 1.0000x reference)
import functools

import jax
import jax.numpy as jnp
from jax import lax
from jax.experimental import pallas as pl
from jax.experimental.pallas import tpu as pltpu
from jax.experimental.pallas import tpu_sc as plsc

D_MODEL = 1024
N_EXPERTS = 8
D_FF = 4096
CAPACITY = 320
T = 2048
C_STRIDE = 352
ECP = N_EXPERTS * C_STRIDE

TOK_BLK = 128
N_TOK_BLK = T // TOK_BLK
FF_BLK = 512
N_FF_BLK = D_FF // FF_BLK

NC, NS = 2, 16
NW = NC * NS


def _route_body(tok_ref, wg_ref, cidx_ref, gidx_ref, wslot_ref,
                cnt_ref, accg_ref, accw_ref):
    i = pl.program_id(0)

    @pl.when(i == 0)
    def _():
        cnt_ref[...] = jnp.zeros_like(cnt_ref)
        accg_ref[...] = jnp.zeros_like(accg_ref)
        accw_ref[...] = jnp.zeros_like(accw_ref)

    x = tok_ref[...]
    logits = jnp.dot(x, wg_ref[...], preferred_element_type=jnp.float32)
    col = lax.broadcasted_iota(jnp.int32, (TOK_BLK, 128), 1)
    valid = col < N_EXPERTS
    lg = jnp.where(valid, logits, -1e30)
    m = jnp.max(lg, axis=1, keepdims=True)
    ex = jnp.where(valid, jnp.exp(lg - m), 0.0)
    probs = ex / jnp.sum(ex, axis=1, keepdims=True)
    wprob = jnp.max(probs, axis=1, keepdims=True)
    eidx = jnp.min(jnp.where((probs == wprob) & valid, col, 127),
                   axis=1, keepdims=True)
    onehot = (col == eidx).astype(jnp.float32)

    row_i = lax.broadcasted_iota(jnp.int32, (TOK_BLK, TOK_BLK), 0)
    col_i = lax.broadcasted_iota(jnp.int32, (TOK_BLK, TOK_BLK), 1)
    tri = (col_i <= row_i).astype(jnp.float32)
    cum = jnp.dot(tri, onehot, preferred_element_type=jnp.float32)
    pos_incl = cum + cnt_ref[0:1, :]
    cnt_ref[0:1, :] = pos_incl[TOK_BLK - 1:TOK_BLK, :]

    posf = jnp.sum(pos_incl * onehot, axis=1, keepdims=True) - 1.0
    keep = posf < float(CAPACITY)
    pos = posf.astype(jnp.int32)
    cidx_ref[...] = eidx * C_STRIDE + jnp.where(keep, pos, C_STRIDE - 1)

    c_iota = lax.broadcasted_iota(jnp.int32, (TOK_BLK, C_STRIDE), 1)
    oh_c = ((c_iota == pos) & keep).astype(jnp.float32)
    tglob = (lax.broadcasted_iota(jnp.int32, (TOK_BLK, 1), 0)
             + i * TOK_BLK).astype(jnp.float32)
    dn = (((0,), (0,)), ((), ()))
    ag = lax.dot_general(onehot * tglob, oh_c, dn,
                         preferred_element_type=jnp.float32)
    aw = lax.dot_general(onehot * wprob, oh_c, dn,
                         preferred_element_type=jnp.float32)
    accg_ref[...] += ag[0:N_EXPERTS, :]
    accw_ref[...] += aw[0:N_EXPERTS, :]
    gidx_ref[...] = accg_ref[...].astype(jnp.int32)
    wslot_ref[...] = accw_ref[...]


def _route(tokens, Wg_pad):
    return pl.pallas_call(
        _route_body,
        grid=(N_TOK_BLK,),
        in_specs=[
            pl.BlockSpec((TOK_BLK, D_MODEL), lambda i: (i, 0)),
            pl.BlockSpec((D_MODEL, 128), lambda i: (0, 0)),
        ],
        out_specs=[
            pl.BlockSpec((TOK_BLK, 1), lambda i: (i, 0)),
            pl.BlockSpec((N_EXPERTS, C_STRIDE), lambda i: (0, 0)),
            pl.BlockSpec((N_EXPERTS, C_STRIDE), lambda i: (0, 0)),
        ],
        out_shape=[
            jax.ShapeDtypeStruct((T, 1), jnp.int32),
            jax.ShapeDtypeStruct((N_EXPERTS, C_STRIDE), jnp.int32),
            jax.ShapeDtypeStruct((N_EXPERTS, C_STRIDE), jnp.float32),
        ],
        scratch_shapes=[
            pltpu.VMEM((8, 128), jnp.float32),
            pltpu.VMEM((N_EXPERTS, C_STRIDE), jnp.float32),
            pltpu.VMEM((N_EXPERTS, C_STRIDE), jnp.float32),
        ],
    )(tokens, Wg_pad)


@functools.cache
def _sc_mesh():
    return plsc.VectorSubcoreMesh(
        core_axis_name="c", subcore_axis_name="s",
        num_cores=NC, num_subcores=NS)


CH = 8


@functools.cache
def _make_sc_gather(n_out):
    bpw = n_out // NW
    n_ch = bpw // CH

    @functools.partial(
        pl.kernel,
        out_type=jax.ShapeDtypeStruct((n_out, D_MODEL), jnp.float32),
        mesh=_sc_mesh(),
        scratch_types=[
            pltpu.VMEM((bpw,), jnp.int32),
            pltpu.VMEM((bpw, D_MODEL), jnp.float32),
            pltpu.SemaphoreType.DMA((n_ch,)),
            pltpu.SemaphoreType.DMA,
        ],
    )
    def sc_gather(table_hbm, idx_hbm, out_hbm, idx_v, rows_v, gsem, ssem):
        wid = lax.axis_index("s") * NC + lax.axis_index("c")
        base = wid * bpw
        pltpu.sync_copy(idx_hbm.at[pl.ds(base, bpw)], idx_v)
        gathers = [
            pltpu.async_copy(
                table_hbm.at[idx_v.at[pl.ds(c * CH, CH)]],
                rows_v.at[pl.ds(c * CH, CH)], gsem.at[c])
            for c in range(n_ch)
        ]
        stores = []
        for c in range(n_ch):
            gathers[c].wait()
            stores.append(pltpu.async_copy(
                rows_v.at[pl.ds(c * CH, CH)],
                out_hbm.at[pl.ds(base + c * CH, CH)], ssem))
        for s in stores:
            s.wait()

    return sc_gather


def _ffn_body(x_ref, w1_ref, b1_ref, w2_ref, b2_ref, ws_ref, out_ref):
    f = pl.program_id(1)
    h = jnp.dot(x_ref[...].astype(jnp.bfloat16),
                w1_ref[0].astype(jnp.bfloat16),
                preferred_element_type=jnp.float32)
    h = jax.nn.gelu(h + b1_ref[0])
    contrib = jnp.dot(h.astype(jnp.bfloat16),
                      w2_ref[0].astype(jnp.bfloat16),
                      preferred_element_type=jnp.float32)

    @pl.when(f == 0)
    def _():
        out_ref[...] = contrib

    @pl.when(f > 0)
    def _():
        out_ref[...] += contrib

    @pl.when(f == N_FF_BLK - 1)
    def _():
        out_ref[...] = (out_ref[...] + b2_ref[0]) * ws_ref[...]


def _ffn(x, W1, b1, W2, b2, wslot_col):
    return pl.pallas_call(
        _ffn_body,
        grid=(N_EXPERTS, N_FF_BLK),
        in_specs=[
            pl.BlockSpec((C_STRIDE, D_MODEL), lambda e, f: (e, 0)),
            pl.BlockSpec((1, D_MODEL, FF_BLK), lambda e, f: (e, 0, f)),
            pl.BlockSpec((1, 1, FF_BLK), lambda e, f: (e, 0, f)),
            pl.BlockSpec((1, FF_BLK, D_MODEL), lambda e, f: (e, f, 0)),
            pl.BlockSpec((1, 1, D_MODEL), lambda e, f: (e, 0, 0)),
            pl.BlockSpec((C_STRIDE, 1), lambda e, f: (e, 0)),
        ],
        out_specs=pl.BlockSpec((C_STRIDE, D_MODEL), lambda e, f: (e, 0)),
        out_shape=jax.ShapeDtypeStruct((ECP, D_MODEL), jnp.float32),
    )(x, W1, b1.reshape(N_EXPERTS, 1, D_FF), W2,
      b2.reshape(N_EXPERTS, 1, D_MODEL), wslot_col)


def kernel(inputs, Wg, W1, b1, W2, b2):
    tokens = inputs.reshape(T, D_MODEL)
    Wg_pad = jnp.pad(Wg, ((0, 0), (0, 128 - N_EXPERTS)))
    cidx, gidx, wslot = _route(tokens, Wg_pad)
    expert_input = _make_sc_gather(ECP)(tokens, gidx.reshape(ECP))
    expert_output = _ffn(expert_input, W1, b1, W2, b2, wslot.reshape(ECP, 1))
    out = _make_sc_gather(T)(expert_output, cidx.reshape(T))
    return out.reshape(inputs.shape)


# speedup vs baseline: 1.4870x; 1.4870x over previous
import functools

import jax
import jax.numpy as jnp
from jax import lax
from jax.experimental import pallas as pl
from jax.experimental.pallas import tpu as pltpu
from jax.experimental.pallas import tpu_sc as plsc

D_MODEL = 1024
N_EXPERTS = 8
D_FF = 4096
CAPACITY = 320
T = 2048
C_STRIDE = 352
ECP = N_EXPERTS * C_STRIDE

TOK_BLK = 256
N_TOK_BLK = T // TOK_BLK
FF_BLK = 1024
N_FF_BLK = D_FF // FF_BLK

NC, NS = 2, 16
NW = NC * NS


def _route_body(tok_ref, wg_ref, cidx_ref, gidx_ref, wslot_ref,
                cnt_ref, accg_ref, accw_ref):
    i = pl.program_id(0)

    @pl.when(i == 0)
    def _():
        cnt_ref[...] = jnp.zeros_like(cnt_ref)
        accg_ref[...] = jnp.zeros_like(accg_ref)
        accw_ref[...] = jnp.zeros_like(accw_ref)

    x = tok_ref[...]
    logits = jnp.dot(x, wg_ref[...], preferred_element_type=jnp.float32)
    col = lax.broadcasted_iota(jnp.int32, (TOK_BLK, 128), 1)
    valid = col < N_EXPERTS
    lg = jnp.where(valid, logits, -1e30)
    m = jnp.max(lg, axis=1, keepdims=True)
    ex = jnp.where(valid, jnp.exp(lg - m), 0.0)
    probs = ex / jnp.sum(ex, axis=1, keepdims=True)
    wprob = jnp.max(probs, axis=1, keepdims=True)
    eidx = jnp.min(jnp.where((probs == wprob) & valid, col, 127),
                   axis=1, keepdims=True)
    onehot = (col == eidx).astype(jnp.float32)

    row_i = lax.broadcasted_iota(jnp.int32, (TOK_BLK, TOK_BLK), 0)
    col_i = lax.broadcasted_iota(jnp.int32, (TOK_BLK, TOK_BLK), 1)
    tri = (col_i <= row_i).astype(jnp.float32)
    cum = jnp.dot(tri, onehot, preferred_element_type=jnp.float32)
    pos_incl = cum + cnt_ref[0:1, :]
    cnt_ref[0:1, :] = pos_incl[TOK_BLK - 1:TOK_BLK, :]

    posf = jnp.sum(pos_incl * onehot, axis=1, keepdims=True) - 1.0
    keep = posf < float(CAPACITY)
    pos = posf.astype(jnp.int32)
    cidx_ref[...] = eidx * C_STRIDE + jnp.where(keep, pos, C_STRIDE - 1)

    c_iota = lax.broadcasted_iota(jnp.int32, (TOK_BLK, C_STRIDE), 1)
    oh_c = ((c_iota == pos) & keep).astype(jnp.float32)
    tglob = (lax.broadcasted_iota(jnp.int32, (TOK_BLK, 1), 0)
             + i * TOK_BLK).astype(jnp.float32)
    dn = (((0,), (0,)), ((), ()))
    ag = lax.dot_general(onehot * tglob, oh_c, dn,
                         preferred_element_type=jnp.float32)
    aw = lax.dot_general(onehot * wprob, oh_c, dn,
                         preferred_element_type=jnp.float32)
    accg_ref[...] += ag[0:N_EXPERTS, :]
    accw_ref[...] += aw[0:N_EXPERTS, :]
    acw = accw_ref[...]
    acg = accg_ref[...].astype(jnp.int32)
    slot = lax.broadcasted_iota(jnp.int32, (N_EXPERTS, C_STRIDE), 0) * C_STRIDE \
        + lax.broadcasted_iota(jnp.int32, (N_EXPERTS, C_STRIDE), 1)
    gidx_ref[...] = jnp.where(acw > 0.0, acg, (slot * 997) & (T - 1))
    wslot_ref[...] = acw


def _route(tokens, Wg_pad):
    return pl.pallas_call(
        _route_body,
        grid=(N_TOK_BLK,),
        in_specs=[
            pl.BlockSpec((TOK_BLK, D_MODEL), lambda i: (i, 0)),
            pl.BlockSpec((D_MODEL, 128), lambda i: (0, 0)),
        ],
        out_specs=[
            pl.BlockSpec((TOK_BLK, 1), lambda i: (i, 0)),
            pl.BlockSpec((N_EXPERTS, C_STRIDE), lambda i: (0, 0)),
            pl.BlockSpec((N_EXPERTS, C_STRIDE), lambda i: (0, 0)),
        ],
        out_shape=[
            jax.ShapeDtypeStruct((T, 1), jnp.int32),
            jax.ShapeDtypeStruct((N_EXPERTS, C_STRIDE), jnp.int32),
            jax.ShapeDtypeStruct((N_EXPERTS, C_STRIDE), jnp.float32),
        ],
        scratch_shapes=[
            pltpu.VMEM((8, 128), jnp.float32),
            pltpu.VMEM((N_EXPERTS, C_STRIDE), jnp.float32),
            pltpu.VMEM((N_EXPERTS, C_STRIDE), jnp.float32),
        ],
    )(tokens, Wg_pad)


@functools.cache
def _sc_mesh():
    return plsc.VectorSubcoreMesh(
        core_axis_name="c", subcore_axis_name="s",
        num_cores=NC, num_subcores=NS)


CH = 8


@functools.cache
def _make_sc_gather(n_out):
    bpw = n_out // NW
    n_ch = bpw // CH

    @functools.partial(
        pl.kernel,
        out_type=jax.ShapeDtypeStruct((n_out, D_MODEL), jnp.float32),
        mesh=_sc_mesh(),
        scratch_types=[
            pltpu.VMEM((bpw,), jnp.int32),
            pltpu.VMEM((bpw, D_MODEL), jnp.float32),
            pltpu.SemaphoreType.DMA((n_ch,)),
            pltpu.SemaphoreType.DMA,
        ],
    )
    def sc_gather(table_hbm, idx_hbm, out_hbm, idx_v, rows_v, gsem, ssem):
        wid = lax.axis_index("s") * NC + lax.axis_index("c")
        base = wid * bpw
        pltpu.sync_copy(idx_hbm.at[pl.ds(base, bpw)], idx_v)
        gathers = [
            pltpu.async_copy(
                table_hbm.at[idx_v.at[pl.ds(c * CH, CH)]],
                rows_v.at[pl.ds(c * CH, CH)], gsem.at[c])
            for c in range(n_ch)
        ]
        stores = []
        for c in range(n_ch):
            gathers[c].wait()
            stores.append(pltpu.async_copy(
                rows_v.at[pl.ds(c * CH, CH)],
                out_hbm.at[pl.ds(base + c * CH, CH)], ssem))
        for s in stores:
            s.wait()

    return sc_gather


def _ffn_body(x_ref, w1_ref, b1_ref, w2_ref, b2_ref, ws_ref, out_ref):
    f = pl.program_id(1)
    h = jnp.dot(x_ref[...].astype(jnp.bfloat16),
                w1_ref[0].astype(jnp.bfloat16),
                preferred_element_type=jnp.float32)
    h = jax.nn.gelu(h + b1_ref[0])
    contrib = jnp.dot(h.astype(jnp.bfloat16),
                      w2_ref[0].astype(jnp.bfloat16),
                      preferred_element_type=jnp.float32)

    @pl.when(f == 0)
    def _():
        out_ref[...] = contrib

    @pl.when(f > 0)
    def _():
        out_ref[...] += contrib

    @pl.when(f == N_FF_BLK - 1)
    def _():
        out_ref[...] = (out_ref[...] + b2_ref[0]) * ws_ref[...]


def _ffn(x, W1, b1, W2, b2, wslot_col):
    return pl.pallas_call(
        _ffn_body,
        grid=(N_EXPERTS, N_FF_BLK),
        in_specs=[
            pl.BlockSpec((C_STRIDE, D_MODEL), lambda e, f: (e, 0)),
            pl.BlockSpec((1, D_MODEL, FF_BLK), lambda e, f: (e, 0, f)),
            pl.BlockSpec((1, 1, FF_BLK), lambda e, f: (e, 0, f)),
            pl.BlockSpec((1, FF_BLK, D_MODEL), lambda e, f: (e, f, 0)),
            pl.BlockSpec((1, 1, D_MODEL), lambda e, f: (e, 0, 0)),
            pl.BlockSpec((C_STRIDE, 1), lambda e, f: (e, 0)),
        ],
        out_specs=pl.BlockSpec((C_STRIDE, D_MODEL), lambda e, f: (e, 0)),
        out_shape=jax.ShapeDtypeStruct((ECP, D_MODEL), jnp.float32),
    )(x, W1, b1.reshape(N_EXPERTS, 1, D_FF), W2,
      b2.reshape(N_EXPERTS, 1, D_MODEL), wslot_col)


def kernel(inputs, Wg, W1, b1, W2, b2):
    tokens = inputs.reshape(T, D_MODEL)
    Wg_pad = jnp.pad(Wg, ((0, 0), (0, 128 - N_EXPERTS)))
    cidx, gidx, wslot = _route(tokens, Wg_pad)
    expert_input = _make_sc_gather(ECP)(tokens, gidx.reshape(ECP))
    expert_output = _ffn(expert_input, W1, b1, W2, b2, wslot.reshape(ECP, 1))
    out = _make_sc_gather(T)(expert_output, cidx.reshape(T))
    return out.reshape(inputs.shape)


# speedup vs baseline: 1.5727x; 1.0576x over previous
import functools

import jax
import jax.numpy as jnp
from jax import lax
from jax.experimental import pallas as pl
from jax.experimental.pallas import tpu as pltpu
from jax.experimental.pallas import tpu_sc as plsc

D_MODEL = 1024
N_EXPERTS = 8
D_FF = 4096
CAPACITY = 320
T = 2048
C_STRIDE = 352
ECP = N_EXPERTS * C_STRIDE

TOK_BLK = 256
N_TOK_BLK = T // TOK_BLK
FF_BLK = 2048
N_FF_BLK = D_FF // FF_BLK

NC, NS = 2, 16
NW = NC * NS


def _route_body(tok_ref, wg_ref, cidx_ref, gidx_ref, wslot_ref,
                cnt_ref, accg_ref, accw_ref):
    i = pl.program_id(0)

    @pl.when(i == 0)
    def _():
        cnt_ref[...] = jnp.zeros_like(cnt_ref)
        accg_ref[...] = jnp.zeros_like(accg_ref)
        accw_ref[...] = jnp.zeros_like(accw_ref)

    x = tok_ref[...]
    logits = jnp.dot(x, wg_ref[...], preferred_element_type=jnp.float32)
    col = lax.broadcasted_iota(jnp.int32, (TOK_BLK, 128), 1)
    valid = col < N_EXPERTS
    lg = jnp.where(valid, logits, -1e30)
    m = jnp.max(lg, axis=1, keepdims=True)
    ex = jnp.where(valid, jnp.exp(lg - m), 0.0)
    probs = ex / jnp.sum(ex, axis=1, keepdims=True)
    wprob = jnp.max(probs, axis=1, keepdims=True)
    eidx = jnp.min(jnp.where((probs == wprob) & valid, col, 127),
                   axis=1, keepdims=True)
    onehot = (col == eidx).astype(jnp.float32)

    row_i = lax.broadcasted_iota(jnp.int32, (TOK_BLK, TOK_BLK), 0)
    col_i = lax.broadcasted_iota(jnp.int32, (TOK_BLK, TOK_BLK), 1)
    tri = (col_i <= row_i).astype(jnp.float32)
    cum = jnp.dot(tri, onehot, preferred_element_type=jnp.float32)
    pos_incl = cum + cnt_ref[0:1, :]
    cnt_ref[0:1, :] = pos_incl[TOK_BLK - 1:TOK_BLK, :]

    posf = jnp.sum(pos_incl * onehot, axis=1, keepdims=True) - 1.0
    keep = posf < float(CAPACITY)
    pos = posf.astype(jnp.int32)
    cidx_ref[...] = eidx * C_STRIDE + jnp.where(keep, pos, C_STRIDE - 1)

    c_iota = lax.broadcasted_iota(jnp.int32, (TOK_BLK, C_STRIDE), 1)
    oh_c = ((c_iota == pos) & keep).astype(jnp.float32)
    tglob = (lax.broadcasted_iota(jnp.int32, (TOK_BLK, 1), 0)
             + i * TOK_BLK).astype(jnp.float32)
    dn = (((0,), (0,)), ((), ()))
    ag = lax.dot_general(onehot * tglob, oh_c, dn,
                         preferred_element_type=jnp.float32)
    aw = lax.dot_general(onehot * wprob, oh_c, dn,
                         preferred_element_type=jnp.float32)
    accg_ref[...] += ag[0:N_EXPERTS, :]
    accw_ref[...] += aw[0:N_EXPERTS, :]
    acw = accw_ref[...]
    acg = accg_ref[...].astype(jnp.int32)
    slot = lax.broadcasted_iota(jnp.int32, (N_EXPERTS, C_STRIDE), 0) * C_STRIDE \
        + lax.broadcasted_iota(jnp.int32, (N_EXPERTS, C_STRIDE), 1)
    gidx_ref[...] = jnp.where(acw > 0.0, acg, (slot * 997) & (T - 1))
    wslot_ref[...] = acw


def _route(tokens, Wg_pad):
    return pl.pallas_call(
        _route_body,
        grid=(N_TOK_BLK,),
        in_specs=[
            pl.BlockSpec((TOK_BLK, D_MODEL), lambda i: (i, 0)),
            pl.BlockSpec((D_MODEL, 128), lambda i: (0, 0)),
        ],
        out_specs=[
            pl.BlockSpec((TOK_BLK, 1), lambda i: (i, 0)),
            pl.BlockSpec((N_EXPERTS, C_STRIDE), lambda i: (0, 0)),
            pl.BlockSpec((N_EXPERTS, C_STRIDE), lambda i: (0, 0)),
        ],
        out_shape=[
            jax.ShapeDtypeStruct((T, 1), jnp.int32),
            jax.ShapeDtypeStruct((N_EXPERTS, C_STRIDE), jnp.int32),
            jax.ShapeDtypeStruct((N_EXPERTS, C_STRIDE), jnp.float32),
        ],
        scratch_shapes=[
            pltpu.VMEM((8, 128), jnp.float32),
            pltpu.VMEM((N_EXPERTS, C_STRIDE), jnp.float32),
            pltpu.VMEM((N_EXPERTS, C_STRIDE), jnp.float32),
        ],
    )(tokens, Wg_pad)


@functools.cache
def _sc_mesh():
    return plsc.VectorSubcoreMesh(
        core_axis_name="c", subcore_axis_name="s",
        num_cores=NC, num_subcores=NS)


CH = 8


@functools.cache
def _make_sc_gather(n_out):
    bpw = n_out // NW
    n_ch = bpw // CH

    @functools.partial(
        pl.kernel,
        out_type=jax.ShapeDtypeStruct((n_out, D_MODEL), jnp.float32),
        mesh=_sc_mesh(),
        scratch_types=[
            pltpu.VMEM((bpw,), jnp.int32),
            pltpu.VMEM((bpw, D_MODEL), jnp.float32),
            pltpu.SemaphoreType.DMA((n_ch,)),
            pltpu.SemaphoreType.DMA,
        ],
    )
    def sc_gather(table_hbm, idx_hbm, out_hbm, idx_v, rows_v, gsem, ssem):
        wid = lax.axis_index("s") * NC + lax.axis_index("c")
        base = wid * bpw
        pltpu.sync_copy(idx_hbm.at[pl.ds(base, bpw)], idx_v)
        gathers = [
            pltpu.async_copy(
                table_hbm.at[idx_v.at[pl.ds(c * CH, CH)]],
                rows_v.at[pl.ds(c * CH, CH)], gsem.at[c])
            for c in range(n_ch)
        ]
        stores = []
        for c in range(n_ch):
            gathers[c].wait()
            stores.append(pltpu.async_copy(
                rows_v.at[pl.ds(c * CH, CH)],
                out_hbm.at[pl.ds(base + c * CH, CH)], ssem))
        for s in stores:
            s.wait()

    return sc_gather


def _ffn_body(x_ref, w1_ref, b1_ref, w2_ref, b2_ref, ws_ref, out_ref):
    f = pl.program_id(1)
    h = jnp.dot(x_ref[...].astype(jnp.bfloat16),
                w1_ref[0].astype(jnp.bfloat16),
                preferred_element_type=jnp.float32)
    h = jax.nn.gelu(h + b1_ref[0])
    contrib = jnp.dot(h.astype(jnp.bfloat16),
                      w2_ref[0].astype(jnp.bfloat16),
                      preferred_element_type=jnp.float32)

    @pl.when(f == 0)
    def _():
        out_ref[...] = contrib

    @pl.when(f > 0)
    def _():
        out_ref[...] += contrib

    @pl.when(f == N_FF_BLK - 1)
    def _():
        out_ref[...] = (out_ref[...] + b2_ref[0]) * ws_ref[...]


def _ffn(x, W1, b1, W2, b2, wslot_col):
    return pl.pallas_call(
        _ffn_body,
        grid=(N_EXPERTS, N_FF_BLK),
        in_specs=[
            pl.BlockSpec((C_STRIDE, D_MODEL), lambda e, f: (e, 0)),
            pl.BlockSpec((1, D_MODEL, FF_BLK), lambda e, f: (e, 0, f)),
            pl.BlockSpec((1, 1, FF_BLK), lambda e, f: (e, 0, f)),
            pl.BlockSpec((1, FF_BLK, D_MODEL), lambda e, f: (e, f, 0)),
            pl.BlockSpec((1, 1, D_MODEL), lambda e, f: (e, 0, 0)),
            pl.BlockSpec((C_STRIDE, 1), lambda e, f: (e, 0)),
        ],
        out_specs=pl.BlockSpec((C_STRIDE, D_MODEL), lambda e, f: (e, 0)),
        out_shape=jax.ShapeDtypeStruct((ECP, D_MODEL), jnp.float32),
    )(x, W1, b1.reshape(N_EXPERTS, 1, D_FF), W2,
      b2.reshape(N_EXPERTS, 1, D_MODEL), wslot_col)


def kernel(inputs, Wg, W1, b1, W2, b2):
    tokens = inputs.reshape(T, D_MODEL)
    Wg_pad = jnp.pad(Wg, ((0, 0), (0, 128 - N_EXPERTS)))
    cidx, gidx, wslot = _route(tokens, Wg_pad)
    expert_input = _make_sc_gather(ECP)(tokens, gidx.reshape(ECP))
    expert_output = _ffn(expert_input, W1, b1, W2, b2, wslot.reshape(ECP, 1))
    out = _make_sc_gather(T)(expert_output, cidx.reshape(T))
    return out.reshape(inputs.shape)


# speedup vs baseline: 1.6029x; 1.0192x over previous
import functools

import jax
import jax.numpy as jnp
from jax import lax
from jax.experimental import pallas as pl
from jax.experimental.pallas import tpu as pltpu
from jax.experimental.pallas import tpu_sc as plsc

D_MODEL = 1024
N_EXPERTS = 8
D_FF = 4096
CAPACITY = 320
T = 2048
C_STRIDE = 352
ECP = N_EXPERTS * C_STRIDE

TOK_BLK = 512
N_TOK_BLK = T // TOK_BLK
FF_BLK = 2048
N_FF_BLK = D_FF // FF_BLK

NC, NS = 2, 16
NW = NC * NS


def _route_body(tok_ref, wg_ref, cidx_ref, gidx_ref, wslot_ref,
                cnt_ref, accg_ref, accw_ref):
    i = pl.program_id(0)

    @pl.when(i == 0)
    def _():
        cnt_ref[...] = jnp.zeros_like(cnt_ref)
        accg_ref[...] = jnp.zeros_like(accg_ref)
        accw_ref[...] = jnp.zeros_like(accw_ref)

    x = tok_ref[...]
    lg = jnp.dot(x, wg_ref[...], preferred_element_type=jnp.float32)
    col = lax.broadcasted_iota(jnp.int32, (TOK_BLK, N_EXPERTS), 1)
    m = jnp.max(lg, axis=1, keepdims=True)
    ex = jnp.exp(lg - m)
    probs = ex / jnp.sum(ex, axis=1, keepdims=True)
    wprob = jnp.max(probs, axis=1, keepdims=True)
    eidx = jnp.min(jnp.where(probs == wprob, col, N_EXPERTS),
                   axis=1, keepdims=True)
    onehot = (col == eidx).astype(jnp.float32)

    row_i = lax.broadcasted_iota(jnp.int32, (TOK_BLK, TOK_BLK), 0)
    col_i = lax.broadcasted_iota(jnp.int32, (TOK_BLK, TOK_BLK), 1)
    tri = (col_i <= row_i).astype(jnp.float32)
    cum = jnp.dot(tri, onehot, preferred_element_type=jnp.float32)
    pos_incl = cum + cnt_ref[0:1, 0:N_EXPERTS]
    cnt_ref[0:1, 0:N_EXPERTS] = pos_incl[TOK_BLK - 1:TOK_BLK, :]

    posf = jnp.sum(pos_incl * onehot, axis=1, keepdims=True) - 1.0
    keep = posf < float(CAPACITY)
    pos = posf.astype(jnp.int32)
    cidx_ref[...] = eidx * C_STRIDE + jnp.where(keep, pos, C_STRIDE - 1)

    c_iota = lax.broadcasted_iota(jnp.int32, (TOK_BLK, C_STRIDE), 1)
    oh_c = ((c_iota == pos) & keep).astype(jnp.float32)
    tglob = (lax.broadcasted_iota(jnp.int32, (TOK_BLK, 1), 0)
             + i * TOK_BLK).astype(jnp.float32)
    dn = (((0,), (0,)), ((), ()))
    ag = lax.dot_general(onehot * tglob, oh_c, dn,
                         preferred_element_type=jnp.float32)
    aw = lax.dot_general(onehot * wprob, oh_c, dn,
                         preferred_element_type=jnp.float32)
    accg_ref[...] += ag
    accw_ref[...] += aw
    acw = accw_ref[...]
    acg = accg_ref[...].astype(jnp.int32)
    slot = lax.broadcasted_iota(jnp.int32, (N_EXPERTS, C_STRIDE), 0) * C_STRIDE \
        + lax.broadcasted_iota(jnp.int32, (N_EXPERTS, C_STRIDE), 1)
    gidx_ref[...] = jnp.where(acw > 0.0, acg, (slot * 997) & (T - 1))
    wslot_ref[...] = acw


def _route(tokens, Wg):
    return pl.pallas_call(
        _route_body,
        grid=(N_TOK_BLK,),
        in_specs=[
            pl.BlockSpec((TOK_BLK, D_MODEL), lambda i: (i, 0)),
            pl.BlockSpec((D_MODEL, N_EXPERTS), lambda i: (0, 0)),
        ],
        out_specs=[
            pl.BlockSpec((TOK_BLK, 1), lambda i: (i, 0)),
            pl.BlockSpec((N_EXPERTS, C_STRIDE), lambda i: (0, 0)),
            pl.BlockSpec((N_EXPERTS, C_STRIDE), lambda i: (0, 0)),
        ],
        out_shape=[
            jax.ShapeDtypeStruct((T, 1), jnp.int32),
            jax.ShapeDtypeStruct((N_EXPERTS, C_STRIDE), jnp.int32),
            jax.ShapeDtypeStruct((N_EXPERTS, C_STRIDE), jnp.float32),
        ],
        scratch_shapes=[
            pltpu.VMEM((8, 128), jnp.float32),
            pltpu.VMEM((N_EXPERTS, C_STRIDE), jnp.float32),
            pltpu.VMEM((N_EXPERTS, C_STRIDE), jnp.float32),
        ],
    )(tokens, Wg)


@functools.cache
def _sc_mesh():
    return plsc.VectorSubcoreMesh(
        core_axis_name="c", subcore_axis_name="s",
        num_cores=NC, num_subcores=NS)


CH = 8


@functools.cache
def _make_sc_gather(n_out):
    bpw = n_out // NW
    n_ch = bpw // CH

    @functools.partial(
        pl.kernel,
        out_type=jax.ShapeDtypeStruct((n_out, D_MODEL), jnp.float32),
        mesh=_sc_mesh(),
        scratch_types=[
            pltpu.VMEM((bpw,), jnp.int32),
            pltpu.VMEM((bpw, D_MODEL), jnp.float32),
            pltpu.SemaphoreType.DMA((n_ch,)),
            pltpu.SemaphoreType.DMA,
        ],
    )
    def sc_gather(table_hbm, idx_hbm, out_hbm, idx_v, rows_v, gsem, ssem):
        wid = lax.axis_index("s") * NC + lax.axis_index("c")
        base = wid * bpw
        pltpu.sync_copy(idx_hbm.at[pl.ds(base, bpw)], idx_v)
        gathers = [
            pltpu.async_copy(
                table_hbm.at[idx_v.at[pl.ds(c * CH, CH)]],
                rows_v.at[pl.ds(c * CH, CH)], gsem.at[c])
            for c in range(n_ch)
        ]
        stores = []
        for c in range(n_ch):
            gathers[c].wait()
            stores.append(pltpu.async_copy(
                rows_v.at[pl.ds(c * CH, CH)],
                out_hbm.at[pl.ds(base + c * CH, CH)], ssem))
        for s in stores:
            s.wait()

    return sc_gather


def _ffn_body(x_ref, w1_ref, b1_ref, w2_ref, b2_ref, ws_ref, out_ref):
    f = pl.program_id(1)
    h = jnp.dot(x_ref[...].astype(jnp.bfloat16),
                w1_ref[0].astype(jnp.bfloat16),
                preferred_element_type=jnp.float32)
    h = jax.nn.gelu(h + b1_ref[0])
    contrib = jnp.dot(h.astype(jnp.bfloat16),
                      w2_ref[0].astype(jnp.bfloat16),
                      preferred_element_type=jnp.float32)

    @pl.when(f == 0)
    def _():
        out_ref[...] = contrib

    @pl.when(f > 0)
    def _():
        out_ref[...] += contrib

    @pl.when(f == N_FF_BLK - 1)
    def _():
        out_ref[...] = (out_ref[...] + b2_ref[0]) * ws_ref[...]


def _ffn(x, W1, b1, W2, b2, wslot_col):
    return pl.pallas_call(
        _ffn_body,
        grid=(N_EXPERTS, N_FF_BLK),
        in_specs=[
            pl.BlockSpec((C_STRIDE, D_MODEL), lambda e, f: (e, 0)),
            pl.BlockSpec((1, D_MODEL, FF_BLK), lambda e, f: (e, 0, f)),
            pl.BlockSpec((1, 1, FF_BLK), lambda e, f: (e, 0, f)),
            pl.BlockSpec((1, FF_BLK, D_MODEL), lambda e, f: (e, f, 0)),
            pl.BlockSpec((1, 1, D_MODEL), lambda e, f: (e, 0, 0)),
            pl.BlockSpec((C_STRIDE, 1), lambda e, f: (e, 0)),
        ],
        out_specs=pl.BlockSpec((C_STRIDE, D_MODEL), lambda e, f: (e, 0)),
        out_shape=jax.ShapeDtypeStruct((ECP, D_MODEL), jnp.float32),
    )(x, W1, b1.reshape(N_EXPERTS, 1, D_FF), W2,
      b2.reshape(N_EXPERTS, 1, D_MODEL), wslot_col)


def kernel(inputs, Wg, W1, b1, W2, b2):
    tokens = inputs.reshape(T, D_MODEL)
    cidx, gidx, wslot = _route(tokens, Wg)
    expert_input = _make_sc_gather(ECP)(tokens, gidx.reshape(ECP))
    expert_output = _ffn(expert_input, W1, b1, W2, b2, wslot.reshape(ECP, 1))
    out = _make_sc_gather(T)(expert_output, cidx.reshape(T))
    return out.reshape(inputs.shape)
